# baseline (speedup 1.0000x reference)
import jax
import jax.numpy as jnp
from jax.experimental import pallas as pl
from jax.experimental.pallas import tpu as pltpu

XR, YR, ZR = 128, 192, 128
VOX = 2.0 / 192.0
SIG = 2.0 / 192.0
INV2S2 = 1.0 / (2.0 * SIG * SIG)
NV = 6890
VPAD = 6912
XBLK = 8


def _axis_weights(vmask, coord_vec, idx, n):
    base = jnp.floor(coord_vec / VOX + (0.5 * n - 0.5))
    center = (idx + (0.5 - 0.5 * n)) * VOX
    d = center - coord_vec
    w = jnp.exp(-(d * d) * INV2S2)
    mask = (idx >= base - 3.0) & (idx <= base + 3.0) & vmask
    return w * mask.astype(jnp.float32)


def _tables_kernel(vx_ref, vy_ref, vz_ref, code_ref,
                   wxt_ref, wyt_ref, bs_ref, bw_ref):
    vmask_l = jax.lax.broadcasted_iota(jnp.int32, (1, VPAD), 1) < NV
    xi = jax.lax.broadcasted_iota(jnp.int32, (XR, 1), 0).astype(jnp.float32)
    wxt_ref[...] = _axis_weights(vmask_l, vx_ref[...], xi, XR)
    yi = jax.lax.broadcasted_iota(jnp.int32, (YR, 1), 0).astype(jnp.float32)
    wyt_ref[...] = _axis_weights(vmask_l, vy_ref[...], yi, YR)

    vmask_s = jax.lax.broadcasted_iota(jnp.int32, (VPAD, 1), 0) < NV
    zi = jax.lax.broadcasted_iota(jnp.int32, (1, ZR), 1).astype(jnp.float32)
    wz = _axis_weights(vmask_s, vz_ref[...], zi, ZR)
    bw_ref[...] = wz

    li = jax.lax.broadcasted_iota(jnp.int32, (1, 3 * ZR), 1)
    zi3 = (li // 3).astype(jnp.float32)
    wz3 = _axis_weights(vmask_s, vz_ref[...], zi3, ZR)
    ci = li % 3
    csel = jnp.where(ci == 0, code_ref[:, 0:1],
                     jnp.where(ci == 1, code_ref[:, 1:2], code_ref[:, 2:3]))
    bs_ref[...] = wz3 * csel


def _accum_kernel(wxt_ref, wyt_ref, bs_ref, bw_ref, occ_ref,
                  osem_ref, ow_ref):
    wyt = wyt_ref[...]
    bs = bs_ref[...].astype(jnp.bfloat16)
    bw = bw_ref[...].astype(jnp.bfloat16)
    erow = jax.lax.broadcasted_iota(jnp.int32, (ZR, 3 * ZR), 0)
    ecol = jax.lax.broadcasted_iota(jnp.int32, (ZR, 3 * ZR), 1)
    emat = (ecol // 3 == erow).astype(jnp.bfloat16)
    for x in range(XBLK):
        row = wxt_ref[x:x + 1, :]
        mt = (wyt * row).astype(jnp.bfloat16)
        acc_s = jax.lax.dot_general(
            mt, bs, (((1,), (0,)), ((), ())),
            preferred_element_type=jnp.float32)
        acc_w = jax.lax.dot_general(
            mt, bw, (((1,), (0,)), ((), ())),
            preferred_element_type=jnp.float32)
        gate = (occ_ref[x] > 1e-3).astype(jnp.bfloat16)
        gate3 = jax.lax.dot_general(
            gate, emat, (((1,), (0,)), ((), ())),
            preferred_element_type=jnp.float32)
        osem_ref[x] = acc_s * gate3
        ow_ref[x] = acc_w * gate.astype(jnp.float32) + 1e-3


def kernel(smpl_vertices, occ_volume, smpl_vertex_code, smpl_face_indices):
    del smpl_face_indices
    pad = VPAD - NV
    verts = jnp.pad(smpl_vertices, ((0, pad), (0, 0)))
    code = jnp.pad(smpl_vertex_code, ((0, pad), (0, 0)))
    vx = verts[:, 0].reshape(1, VPAD)
    vy = verts[:, 1].reshape(1, VPAD)
    vz = verts[:, 2].reshape(VPAD, 1)

    wxt, wyt, bs, bw = pl.pallas_call(
        _tables_kernel,
        out_shape=[
            jax.ShapeDtypeStruct((XR, VPAD), jnp.float32),
            jax.ShapeDtypeStruct((YR, VPAD), jnp.float32),
            jax.ShapeDtypeStruct((VPAD, 3 * ZR), jnp.float32),
            jax.ShapeDtypeStruct((VPAD, ZR), jnp.float32),
        ],
    )(vx, vy, vz, code)

    osem, ow = pl.pallas_call(
        _accum_kernel,
        grid=(XR // XBLK,),
        in_specs=[
            pl.BlockSpec((XBLK, VPAD), lambda i: (i, 0)),
            pl.BlockSpec((YR, VPAD), lambda i: (0, 0)),
            pl.BlockSpec((VPAD, 3 * ZR), lambda i: (0, 0)),
            pl.BlockSpec((VPAD, ZR), lambda i: (0, 0)),
            pl.BlockSpec((XBLK, YR, ZR), lambda i: (i, 0, 0)),
        ],
        out_specs=[
            pl.BlockSpec((XBLK, YR, 3 * ZR), lambda i: (i, 0, 0)),
            pl.BlockSpec((XBLK, YR, ZR), lambda i: (i, 0, 0)),
        ],
        out_shape=[
            jax.ShapeDtypeStruct((XR, YR, 3 * ZR), jnp.float32),
            jax.ShapeDtypeStruct((XR, YR, ZR), jnp.float32),
        ],
    )(wxt, wyt, bs, bw, occ_volume)

    semantic_volume = osem.reshape(XR, YR, ZR, 3)
    weight_sum_volume = ow
    return semantic_volume, weight_sum_volume


# speedup vs baseline: 1.3159x; 1.3159x over previous
import jax
import jax.numpy as jnp
from jax.experimental import pallas as pl
from jax.experimental.pallas import tpu as pltpu

XR, YR, ZR = 128, 192, 128
VOX = 2.0 / 192.0
SIG = 2.0 / 192.0
INV2S2 = 1.0 / (2.0 * SIG * SIG)
NV = 6890
VPAD = 6912
XBLK = 8


def _axis_weights(vmask, coord_vec, idx, n):
    base = jnp.floor(coord_vec / VOX + (0.5 * n - 0.5))
    center = (idx + (0.5 - 0.5 * n)) * VOX
    d = center - coord_vec
    w = jnp.exp(-(d * d) * INV2S2)
    mask = (idx >= base - 3.0) & (idx <= base + 3.0) & vmask
    return w * mask.astype(jnp.float32)


def _tables_kernel(vx_ref, vy_ref, vz_ref, code_ref,
                   wxt_ref, wyt_ref, b_ref):
    vmask_l = jax.lax.broadcasted_iota(jnp.int32, (1, VPAD), 1) < NV
    xi = jax.lax.broadcasted_iota(jnp.int32, (XR, 1), 0).astype(jnp.float32)
    wxt_ref[...] = _axis_weights(vmask_l, vx_ref[...], xi, XR)
    yi = jax.lax.broadcasted_iota(jnp.int32, (YR, 1), 0).astype(jnp.float32)
    wyt_ref[...] = _axis_weights(vmask_l, vy_ref[...], yi, YR)

    vmask_s = jax.lax.broadcasted_iota(jnp.int32, (VPAD, 1), 0) < NV
    zi = jax.lax.broadcasted_iota(jnp.int32, (1, ZR), 1).astype(jnp.float32)
    wz = _axis_weights(vmask_s, vz_ref[...], zi, ZR)
    b_ref[:, 3 * ZR:] = wz

    li = jax.lax.broadcasted_iota(jnp.int32, (1, 3 * ZR), 1)
    zi3 = (li // 3).astype(jnp.float32)
    wz3 = _axis_weights(vmask_s, vz_ref[...], zi3, ZR)
    ci = li % 3
    csel = jnp.where(ci == 0, code_ref[:, 0:1],
                     jnp.where(ci == 1, code_ref[:, 1:2], code_ref[:, 2:3]))
    b_ref[:, :3 * ZR] = wz3 * csel


def _accum_kernel(wxt_ref, wyt_ref, b_ref, occ_ref, osem_ref, ow_ref):
    wyt = wyt_ref[...]
    bmat = b_ref[...].astype(jnp.bfloat16)
    erow = jax.lax.broadcasted_iota(jnp.int32, (ZR, 3 * ZR), 0)
    ecol = jax.lax.broadcasted_iota(jnp.int32, (ZR, 3 * ZR), 1)
    emat = (ecol // 3 == erow).astype(jnp.bfloat16)
    for x in range(XBLK):
        row = wxt_ref[x:x + 1, :]
        mt = (wyt * row).astype(jnp.bfloat16)
        acc = jax.lax.dot_general(
            mt, bmat, (((1,), (0,)), ((), ())),
            preferred_element_type=jnp.float32)
        gate = (occ_ref[x] > 1e-3).astype(jnp.bfloat16)
        gate3 = jax.lax.dot_general(
            gate, emat, (((1,), (0,)), ((), ())),
            preferred_element_type=jnp.float32)
        osem_ref[x] = acc[:, :3 * ZR] * gate3
        ow_ref[x] = acc[:, 3 * ZR:] * gate.astype(jnp.float32) + 1e-3


def kernel(smpl_vertices, occ_volume, smpl_vertex_code, smpl_face_indices):
    del smpl_face_indices
    pad = VPAD - NV
    verts = jnp.pad(smpl_vertices, ((0, pad), (0, 0)))
    code = jnp.pad(smpl_vertex_code, ((0, pad), (0, 0)))
    vx = verts[:, 0].reshape(1, VPAD)
    vy = verts[:, 1].reshape(1, VPAD)
    vz = verts[:, 2].reshape(VPAD, 1)

    wxt, wyt, bmat = pl.pallas_call(
        _tables_kernel,
        out_shape=[
            jax.ShapeDtypeStruct((XR, VPAD), jnp.float32),
            jax.ShapeDtypeStruct((YR, VPAD), jnp.float32),
            jax.ShapeDtypeStruct((VPAD, 4 * ZR), jnp.float32),
        ],
    )(vx, vy, vz, code)

    osem, ow = pl.pallas_call(
        _accum_kernel,
        grid=(XR // XBLK,),
        in_specs=[
            pl.BlockSpec((XBLK, VPAD), lambda i: (i, 0)),
            pl.BlockSpec((YR, VPAD), lambda i: (0, 0)),
            pl.BlockSpec((VPAD, 4 * ZR), lambda i: (0, 0)),
            pl.BlockSpec((XBLK, YR, ZR), lambda i: (i, 0, 0)),
        ],
        out_specs=[
            pl.BlockSpec((XBLK, YR, 3 * ZR), lambda i: (i, 0, 0)),
            pl.BlockSpec((XBLK, YR, ZR), lambda i: (i, 0, 0)),
        ],
        out_shape=[
            jax.ShapeDtypeStruct((XR, YR, 3 * ZR), jnp.float32),
            jax.ShapeDtypeStruct((XR, YR, ZR), jnp.float32),
        ],
    )(wxt, wyt, bmat, occ_volume)

    semantic_volume = osem.reshape(XR, YR, ZR, 3)
    weight_sum_volume = ow
    return semantic_volume, weight_sum_volume


# speedup vs baseline: 1.3161x; 1.0002x over previous
import jax
import jax.numpy as jnp
from jax.experimental import pallas as pl
from jax.experimental.pallas import tpu as pltpu

XR, YR, ZR = 128, 192, 128
VOX = 2.0 / 192.0
SIG = 2.0 / 192.0
INV2S2 = 1.0 / (2.0 * SIG * SIG)
NV = 6890
VPAD = 6912
XBLK = 8


def _axis_weights(vmask, coord_vec, idx, n):
    base = jnp.floor(coord_vec / VOX + (0.5 * n - 0.5))
    center = (idx + (0.5 - 0.5 * n)) * VOX
    d = center - coord_vec
    w = jnp.exp(-(d * d) * INV2S2)
    mask = (idx >= base - 3.0) & (idx <= base + 3.0) & vmask
    return w * mask.astype(jnp.float32)


def _tables_kernel(vx_ref, vy_ref, vz_ref, code_ref,
                   wxt_ref, wyt_ref, b_ref):
    vmask_l = jax.lax.broadcasted_iota(jnp.int32, (1, VPAD), 1) < NV
    xi = jax.lax.broadcasted_iota(jnp.int32, (XR, 1), 0).astype(jnp.float32)
    wxt_ref[...] = _axis_weights(vmask_l, vx_ref[...], xi, XR)
    yi = jax.lax.broadcasted_iota(jnp.int32, (YR, 1), 0).astype(jnp.float32)
    wyt_ref[...] = _axis_weights(vmask_l, vy_ref[...], yi, YR)

    vmask_s = jax.lax.broadcasted_iota(jnp.int32, (VPAD, 1), 0) < NV
    zi = jax.lax.broadcasted_iota(jnp.int32, (1, ZR), 1).astype(jnp.float32)
    wz = _axis_weights(vmask_s, vz_ref[...], zi, ZR)
    b_ref[:, 3 * ZR:] = wz

    li = jax.lax.broadcasted_iota(jnp.int32, (1, 3 * ZR), 1)
    zi3 = (li // 3).astype(jnp.float32)
    wz3 = _axis_weights(vmask_s, vz_ref[...], zi3, ZR)
    ci = li % 3
    csel = jnp.where(ci == 0, code_ref[:, 0:1],
                     jnp.where(ci == 1, code_ref[:, 1:2], code_ref[:, 2:3]))
    b_ref[:, :3 * ZR] = wz3 * csel


def _accum_kernel(wxt_ref, wyt_ref, b_ref, occ_ref, osem_ref, ow_ref):
    wyt = wyt_ref[...]
    bmat = b_ref[...].astype(jnp.bfloat16)
    erow = jax.lax.broadcasted_iota(jnp.int32, (ZR, 3 * ZR), 0)
    ecol = jax.lax.broadcasted_iota(jnp.int32, (ZR, 3 * ZR), 1)
    emat = (ecol // 3 == erow).astype(jnp.bfloat16)
    for x in range(XBLK):
        row = wxt_ref[x:x + 1, :]
        mt = (wyt * row).astype(jnp.bfloat16)
        acc = jax.lax.dot_general(
            mt, bmat, (((1,), (0,)), ((), ())),
            preferred_element_type=jnp.float32)
        gate = (occ_ref[x] > 1e-3).astype(jnp.bfloat16)
        gate3 = jax.lax.dot_general(
            gate, emat, (((1,), (0,)), ((), ())),
            preferred_element_type=jnp.float32)
        osem_ref[x] = acc[:, :3 * ZR] * gate3
        ow_ref[x] = acc[:, 3 * ZR:] * gate.astype(jnp.float32) + 1e-3


def kernel(smpl_vertices, occ_volume, smpl_vertex_code, smpl_face_indices):
    del smpl_face_indices
    pad = VPAD - NV
    verts = jnp.pad(smpl_vertices, ((0, pad), (0, 0)))
    code = jnp.pad(smpl_vertex_code, ((0, pad), (0, 0)))
    vx = verts[:, 0].reshape(1, VPAD)
    vy = verts[:, 1].reshape(1, VPAD)
    vz = verts[:, 2].reshape(VPAD, 1)

    wxt, wyt, bmat = pl.pallas_call(
        _tables_kernel,
        out_shape=[
            jax.ShapeDtypeStruct((XR, VPAD), jnp.float32),
            jax.ShapeDtypeStruct((YR, VPAD), jnp.float32),
            jax.ShapeDtypeStruct((VPAD, 4 * ZR), jnp.float32),
        ],
    )(vx, vy, vz, code)

    osem, ow = pl.pallas_call(
        _accum_kernel,
        grid=(XR // XBLK,),
        in_specs=[
            pl.BlockSpec((XBLK, VPAD), lambda i: (i, 0)),
            pl.BlockSpec((YR, VPAD), lambda i: (0, 0)),
            pl.BlockSpec((VPAD, 4 * ZR), lambda i: (0, 0)),
            pl.BlockSpec((XBLK, YR, ZR), lambda i: (i, 0, 0)),
        ],
        out_specs=[
            pl.BlockSpec((XBLK, YR, 3 * ZR), lambda i: (i, 0, 0)),
            pl.BlockSpec((XBLK, YR, ZR), lambda i: (i, 0, 0)),
        ],
        out_shape=[
            jax.ShapeDtypeStruct((XR, YR, 3 * ZR), jnp.float32),
            jax.ShapeDtypeStruct((XR, YR, ZR), jnp.float32),
        ],
        compiler_params=pltpu.CompilerParams(
            dimension_semantics=("parallel",)),
    )(wxt, wyt, bmat, occ_volume)

    semantic_volume = osem.reshape(XR, YR, ZR, 3)
    weight_sum_volume = ow
    return semantic_volume, weight_sum_volume
